# initial kernel scaffold (unmeasured)
import jax
import jax.numpy as jnp
from jax import lax
from jax.experimental import pallas as pl
from jax.experimental.pallas import tpu as pltpu

N_DEV = 32


def kernel(x, Win0, Wout0, Win1, Wout1, Win2, Wout2):
    b, d = x.shape
    rows = b // N_DEV

    def body(x_ref, win0, wout0, win1, wout1, win2, wout2, out_ref,
             part_ref, rs_buf, red_ref, xnext_ref,
             rs_send, rs_recv, ag_send, ag_recv):
        me = lax.axis_index("i")

        def reduce_scatter(part_val):
            part_ref[...] = part_val
            rdmas = []
            for off in range(1, N_DEV):
                dst = lax.rem(me + off, N_DEV)
                rdma = pltpu.make_async_remote_copy(
                    src_ref=part_ref.at[pl.ds(dst * rows, rows), :],
                    dst_ref=rs_buf.at[off],
                    send_sem=rs_send.at[off],
                    recv_sem=rs_recv.at[off],
                    device_id=(dst,),
                    device_id_type=pl.DeviceIdType.MESH,
                )
                rdma.start()
                rdmas.append(rdma)
            acc = part_ref[pl.ds(me * rows, rows), :]
            for off, rdma in zip(range(1, N_DEV), rdmas):
                rdma.wait_recv()
                acc = acc + rs_buf[off]
            for rdma in rdmas:
                rdma.wait_send()
            return acc

        def all_gather(red):
            red_ref[...] = red
            rdmas = []
            for off in range(1, N_DEV):
                dst = lax.rem(me + off, N_DEV)
                rdma = pltpu.make_async_remote_copy(
                    src_ref=red_ref,
                    dst_ref=xnext_ref.at[pl.ds(me * rows, rows), :],
                    send_sem=ag_send.at[off],
                    recv_sem=ag_recv.at[off],
                    device_id=(dst,),
                    device_id_type=pl.DeviceIdType.MESH,
                )
                rdma.start()
                rdmas.append(rdma)
            xnext_ref[pl.ds(me * rows, rows), :] = red
            for rdma in rdmas:
                rdma.wait_recv()
            for rdma in rdmas:
                rdma.wait_send()
            return xnext_ref[...]

        def layer(xv, win, wout):
            h = jnp.maximum(
                jnp.dot(xv, win[...], preferred_element_type=jnp.float32), 0.0
            )
            return jnp.dot(h, wout[...], preferred_element_type=jnp.float32)

        xv = x_ref[...]
        red = reduce_scatter(layer(xv, win0, wout0))
        xv = all_gather(red)
        red = reduce_scatter(layer(xv, win1, wout1))
        xv = all_gather(red)
        red = reduce_scatter(layer(xv, win2, wout2))
        out_ref[...] = red

    return pl.pallas_call(
        body,
        out_shape=jax.ShapeDtypeStruct((rows, d), jnp.float32),
        in_specs=[pl.BlockSpec(memory_space=pltpu.VMEM)] * 7,
        out_specs=pl.BlockSpec(memory_space=pltpu.VMEM),
        scratch_shapes=[
            pltpu.VMEM((b, d), jnp.float32),
            pltpu.VMEM((N_DEV, rows, d), jnp.float32),
            pltpu.VMEM((rows, d), jnp.float32),
            pltpu.VMEM((b, d), jnp.float32),
            pltpu.SemaphoreType.DMA((N_DEV,)),
            pltpu.SemaphoreType.DMA((N_DEV,)),
            pltpu.SemaphoreType.DMA((N_DEV,)),
            pltpu.SemaphoreType.DMA((N_DEV,)),
        ],
    )(x, Win0, Wout0, Win1, Wout1, Win2, Wout2)


# baseline (device time: 65146 ns/iter reference)
import jax
import jax.numpy as jnp
from jax import lax
from jax.experimental import pallas as pl
from jax.experimental.pallas import tpu as pltpu

N_DEV = 32


def kernel(x, Win0, Wout0, Win1, Wout1, Win2, Wout2):
    b, d = x.shape
    rows = b // N_DEV

    def body(x_ref, win0, wout0, win1, wout1, win2, wout2, out_ref,
             part_ref, rs_buf, red_ref, xnext_ref,
             rs_send, rs_recv, ag_send, ag_recv, loc_sem):
        me = lax.axis_index("i")

        def reduce_scatter(part_val):
            part_ref[...] = part_val
            self_copy = pltpu.make_async_copy(
                part_ref.at[pl.ds(me * rows, rows), :], rs_buf.at[0], loc_sem
            )
            self_copy.start()
            rdmas = []
            for off in range(1, N_DEV):
                dst = lax.rem(me + off, N_DEV)
                rdma = pltpu.make_async_remote_copy(
                    src_ref=part_ref.at[pl.ds(dst * rows, rows), :],
                    dst_ref=rs_buf.at[off],
                    send_sem=rs_send.at[off],
                    recv_sem=rs_recv.at[off],
                    device_id=(dst,),
                    device_id_type=pl.DeviceIdType.MESH,
                )
                rdma.start()
                rdmas.append(rdma)
            self_copy.wait()
            acc = rs_buf[0]
            for off, rdma in zip(range(1, N_DEV), rdmas):
                rdma.wait_recv()
                acc = acc + rs_buf[off]
            for rdma in rdmas:
                rdma.wait_send()
            return acc

        def all_gather(red):
            red_ref[...] = red
            self_copy = pltpu.make_async_copy(
                red_ref, xnext_ref.at[pl.ds(me * rows, rows), :], loc_sem
            )
            self_copy.start()
            rdmas = []
            for off in range(1, N_DEV):
                dst = lax.rem(me + off, N_DEV)
                rdma = pltpu.make_async_remote_copy(
                    src_ref=red_ref,
                    dst_ref=xnext_ref.at[pl.ds(me * rows, rows), :],
                    send_sem=ag_send.at[off],
                    recv_sem=ag_recv.at[off],
                    device_id=(dst,),
                    device_id_type=pl.DeviceIdType.MESH,
                )
                rdma.start()
                rdmas.append(rdma)
            self_copy.wait()
            for rdma in rdmas:
                rdma.wait_recv()
            for rdma in rdmas:
                rdma.wait_send()
            return xnext_ref[...]

        def layer(xv, win, wout):
            h = jnp.maximum(
                jnp.dot(xv, win[...], preferred_element_type=jnp.float32), 0.0
            )
            return jnp.dot(h, wout[...], preferred_element_type=jnp.float32)

        xv = x_ref[...]
        red = reduce_scatter(layer(xv, win0, wout0))
        xv = all_gather(red)
        red = reduce_scatter(layer(xv, win1, wout1))
        xv = all_gather(red)
        red = reduce_scatter(layer(xv, win2, wout2))
        out_ref[...] = red

    return pl.pallas_call(
        body,
        out_shape=jax.ShapeDtypeStruct((rows, d), jnp.float32),
        in_specs=[pl.BlockSpec(memory_space=pltpu.VMEM)] * 7,
        out_specs=pl.BlockSpec(memory_space=pltpu.VMEM),
        scratch_shapes=[
            pltpu.VMEM((b, d), jnp.float32),
            pltpu.VMEM((N_DEV, rows, d), jnp.float32),
            pltpu.VMEM((rows, d), jnp.float32),
            pltpu.VMEM((b, d), jnp.float32),
            pltpu.SemaphoreType.DMA((N_DEV,)),
            pltpu.SemaphoreType.DMA((N_DEV,)),
            pltpu.SemaphoreType.DMA((N_DEV,)),
            pltpu.SemaphoreType.DMA((N_DEV,)),
            pltpu.SemaphoreType.DMA,
        ],
        compiler_params=pltpu.CompilerParams(
            vmem_limit_bytes=100 * 1024 * 1024,
        ),
    )(x, Win0, Wout0, Win1, Wout1, Win2, Wout2)
